# Initial kernel scaffold; baseline (speedup 1.0000x reference)
import jax
import jax.numpy as jnp
from jax.experimental import pallas as pl


def kernel(x, edge_index, W_ih, b_ih, W_hh, b_hh, Wg, bg, Wm, Wp, bp, Wc1, bc1, Wc2, bc2):
    raise NotImplementedError("write your pallas kernel here")


# speedup vs baseline: 4.1891x; 4.1891x over previous
import functools

import jax
import jax.numpy as jnp
from jax import lax
from jax.experimental import pallas as pl
from jax.experimental.pallas import tpu as pltpu
from jax.experimental.pallas import tpu_sc as plsc

N_NODES = 10000
N_EDGES = 160000
VHS = 100
VW = 128
NVT = 3
CHS = 30

NC = 2
NS = 16
NW = NC * NS
CHUNK = 128
NCHUNK = (N_EDGES + NW * CHUNK - 1) // (NW * CHUNK)
E_PAD = NW * NCHUNK * CHUNK
ROWS_PER_TILE = 632
ACC_ROWS = NS * ROWS_PER_TILE

BR = 1000
GRID = N_NODES // BR


def _sc_scatter_body(m_hbm, src_hbm, dst_hbm, zeros_hbm, part_hbm,
                     src_v, dst_v, rows_v, acc, sem):
    c = lax.axis_index("c")
    s = lax.axis_index("s")
    wid = s * NC + c

    pltpu.sync_copy(src_hbm.at[wid], src_v)
    pltpu.sync_copy(dst_hbm.at[wid], dst_v)
    pltpu.sync_copy(zeros_hbm, acc.at[pl.ds(s * ROWS_PER_TILE, ROWS_PER_TILE)])
    plsc.subcore_barrier()

    def body(j, carry):
        pltpu.async_copy(m_hbm.at[src_v.at[j]], rows_v, sem).wait()
        pltpu.sync_copy(rows_v, acc.at[dst_v.at[j]], add=True)
        return carry

    lax.fori_loop(0, NCHUNK, body, 0)
    plsc.subcore_barrier()
    pltpu.sync_copy(acc.at[pl.ds(s * ROWS_PER_TILE, ROWS_PER_TILE)],
                    part_hbm.at[c, pl.ds(s * ROWS_PER_TILE, ROWS_PER_TILE)])


@functools.cache
def _make_sc_scatter():
    return functools.partial(
        pl.kernel,
        out_type=jax.ShapeDtypeStruct((NC, ACC_ROWS, VW), jnp.float32),
        mesh=plsc.VectorSubcoreMesh(core_axis_name="c", subcore_axis_name="s",
                                    num_cores=NC, num_subcores=NS),
        scratch_types=[
            pltpu.VMEM((NCHUNK, CHUNK), jnp.int32),
            pltpu.VMEM((NCHUNK, CHUNK), jnp.int32),
            pltpu.VMEM((CHUNK, VW), jnp.float32),
            pltpu.VMEM_SHARED((ACC_ROWS, VW), jnp.float32),
            pltpu.SemaphoreType.DMA,
        ],
    )(_sc_scatter_body)


def _dot(a, b):
    return jnp.dot(a, b, preferred_element_type=jnp.float32)


def _msg_and_proj(h, wg, bg, wm, wp, bp, m_ref, inp_ref):
    m = jax.nn.sigmoid(_dot(h, wg[...]) + bg[...]) * _dot(h, wm[...])
    m_ref[...] = jnp.concatenate(
        [m, jnp.zeros((m.shape[0], VW - VHS), jnp.float32)], axis=1)
    inp_ref[...] = _dot(h, wp[...]) + bp[...]


def _tc_round0_body(x_ref, wir, wiz, win, bir, biz, bin_, bhr, bhz, bhn,
                    wg, bg, wm, wp, bp, m_ref, inp_ref):
    x = x_ref[...]
    ir = _dot(x, wir[...]) + bir[...]
    iz = _dot(x, wiz[...]) + biz[...]
    inn = _dot(x, win[...]) + bin_[...]
    r = jax.nn.sigmoid(ir + bhr[...])
    z = jax.nn.sigmoid(iz + bhz[...])
    n = jnp.tanh(inn + r * bhn[...])
    h = (1.0 - z) * n
    _msg_and_proj(h, wg, bg, wm, wp, bp, m_ref, inp_ref)


def _gru_from_parts(inp_ref, p0_ref, p1_ref, wir, wiz, win, bir, biz, bin_,
                    whr, whz, whn, bhr, bhz, bhn):
    ps = p0_ref[0] + p1_ref[0]
    inp = inp_ref[...]
    ir = _dot(inp, wir[...]) + bir[...]
    iz = _dot(inp, wiz[...]) + biz[...]
    inn = _dot(inp, win[...]) + bin_[...]
    hr = _dot(ps, whr[...]) + bhr[...]
    hz = _dot(ps, whz[...]) + bhz[...]
    hn = _dot(ps, whn[...]) + bhn[...]
    r = jax.nn.sigmoid(ir + hr)
    z = jax.nn.sigmoid(iz + hz)
    n = jnp.tanh(inn + r * hn)
    return (1.0 - z) * n + z * ps[:, :VHS]


def _tc_round_body(inp_ref, p0_ref, p1_ref, wir, wiz, win, bir, biz, bin_,
                   whr, whz, whn, bhr, bhz, bhn, wg, bg, wm, wp, bp,
                   m_ref, inp2_ref):
    h = _gru_from_parts(inp_ref, p0_ref, p1_ref, wir, wiz, win, bir, biz,
                        bin_, whr, whz, whn, bhr, bhz, bhn)
    _msg_and_proj(h, wg, bg, wm, wp, bp, m_ref, inp2_ref)


def _tc_final_body(inp_ref, p0_ref, p1_ref, wir, wiz, win, bir, biz, bin_,
                   whr, whz, whn, bhr, bhz, bhn, wc1, bc1, wc2, bc2, out_ref):
    h = _gru_from_parts(inp_ref, p0_ref, p1_ref, wir, wiz, win, bir, biz,
                        bin_, whr, whz, whn, bhr, bhz, bhn)
    hid = jax.nn.relu(_dot(h, wc1[...]) + bc1[...])
    out_ref[...] = jax.nn.sigmoid(_dot(hid, wc2[...]) + bc2[...])


def _full(shape):
    return pl.BlockSpec(shape, lambda i: (0,) * len(shape))


def _rows(width):
    return pl.BlockSpec((BR, width), lambda i: (i, 0))


def _part_spec(core):
    return pl.BlockSpec((1, BR, VW), lambda i, c=core: (c, i, 0))


def kernel(x, edge_index, W_ih, b_ih, W_hh, b_hh, Wg, bg, Wm, Wp, bp,
           Wc1, bc1, Wc2, bc2):
    f32 = jnp.float32

    wihT = W_ih.T
    wir, wiz, win = (wihT[:, :VHS], wihT[:, VHS:2 * VHS], wihT[:, 2 * VHS:])
    whhT = W_hh.T
    pad_w = ((0, VW - VHS), (0, 0))
    whr = jnp.pad(whhT[:, :VHS], pad_w)
    whz = jnp.pad(whhT[:, VHS:2 * VHS], pad_w)
    whn = jnp.pad(whhT[:, 2 * VHS:], pad_w)
    bir, biz, bin_ = (b_ih[:VHS].reshape(1, VHS),
                      b_ih[VHS:2 * VHS].reshape(1, VHS),
                      b_ih[2 * VHS:].reshape(1, VHS))
    bhr, bhz, bhn = (b_hh[:VHS].reshape(1, VHS),
                     b_hh[VHS:2 * VHS].reshape(1, VHS),
                     b_hh[2 * VHS:].reshape(1, VHS))
    wg, wm, wp = Wg.T, Wm.T, Wp.T
    bg2, bp2 = bg.reshape(1, VHS), bp.reshape(1, NVT)
    wc1, wc2 = Wc1.T, Wc2.T
    bc12, bc22 = bc1.reshape(1, CHS), bc2.reshape(1, 1)

    pad_e = E_PAD - N_EDGES
    src3 = jnp.concatenate(
        [edge_index[0], jnp.zeros((pad_e,), jnp.int32)]).reshape(
            NW, NCHUNK, CHUNK)
    dst3 = jnp.concatenate(
        [edge_index[1], jnp.full((pad_e,), N_NODES, jnp.int32)]).reshape(
            NW, NCHUNK, CHUNK)
    zeros_tile = jnp.zeros((ROWS_PER_TILE, VW), f32)

    gru_w = (wir, wiz, win, bir, biz, bin_)
    gru_h = (whr, whz, whn, bhr, bhz, bhn)
    msg_w = (wg, bg2, wm, wp, bp2)

    gru_w_specs = [_full((NVT, VHS))] * 3 + [_full((1, VHS))] * 3
    gru_h_specs = [_full((VW, VHS))] * 3 + [_full((1, VHS))] * 3
    msg_specs = [_full((VHS, VHS)), _full((1, VHS)), _full((VHS, VHS)),
                 _full((VHS, NVT)), _full((1, NVT))]

    m_shape = jax.ShapeDtypeStruct((N_NODES, VW), f32)
    inp_shape = jax.ShapeDtypeStruct((N_NODES, NVT), f32)

    m1, inp1 = pl.pallas_call(
        _tc_round0_body,
        grid=(GRID,),
        in_specs=[_rows(NVT)] + gru_w_specs + gru_h_specs[3:] + msg_specs,
        out_specs=[_rows(VW), _rows(NVT)],
        out_shape=[m_shape, inp_shape],
    )(x, *gru_w, *gru_h[3:], *msg_w)

    sc_scatter = _make_sc_scatter()
    parts1 = sc_scatter(m1, src3, dst3, zeros_tile)
    m2, inp2 = pl.pallas_call(
        _tc_round_body,
        grid=(GRID,),
        in_specs=([_rows(NVT), _part_spec(0), _part_spec(1)]
                  + gru_w_specs + gru_h_specs + msg_specs),
        out_specs=[_rows(VW), _rows(NVT)],
        out_shape=[m_shape, inp_shape],
    )(inp1, parts1, parts1, *gru_w, *gru_h, *msg_w)

    parts2 = sc_scatter(m2, src3, dst3, zeros_tile)
    out = pl.pallas_call(
        _tc_final_body,
        grid=(GRID,),
        in_specs=([_rows(NVT), _part_spec(0), _part_spec(1)]
                  + gru_w_specs + gru_h_specs
                  + [_full((VHS, CHS)), _full((1, CHS)),
                     _full((CHS, 1)), _full((1, 1))]),
        out_specs=[_rows(1)],
        out_shape=[jax.ShapeDtypeStruct((N_NODES, 1), f32)],
    )(inp2, parts2, parts2, *gru_w, *gru_h, wc1, bc12, wc2, bc22)[0]

    return out


# speedup vs baseline: 4.4717x; 1.0675x over previous
import functools

import jax
import jax.numpy as jnp
from jax import lax
from jax.experimental import pallas as pl
from jax.experimental.pallas import tpu as pltpu
from jax.experimental.pallas import tpu_sc as plsc

N_NODES = 10000
N_EDGES = 160000
VHS = 100
VW = 128
NVT = 3
CHS = 30

NC = 2
NS = 16
NW = NC * NS
CHUNK = 128
NCHUNK = (N_EDGES + NW * CHUNK - 1) // (NW * CHUNK)
E_PAD = NW * NCHUNK * CHUNK
ROWS_PER_TILE = 632
ACC_ROWS = NS * ROWS_PER_TILE

BR = 1000
GRID = N_NODES // BR


def _sc_scatter_body(m_hbm, src_hbm, dst_hbm, zeros_hbm, part_hbm,
                     src_v, dst_v, rows0, rows1, acc, sem0, sem1):
    c = lax.axis_index("c")
    s = lax.axis_index("s")
    wid = s * NC + c

    pltpu.sync_copy(src_hbm.at[wid], src_v)
    pltpu.sync_copy(dst_hbm.at[wid], dst_v)
    pltpu.sync_copy(zeros_hbm, acc.at[pl.ds(s * ROWS_PER_TILE, ROWS_PER_TILE)])
    plsc.subcore_barrier()

    pltpu.async_copy(m_hbm.at[src_v.at[0]], rows0, sem0)

    def wait_gather(buf, sem):
        pltpu.make_async_copy(m_hbm.at[src_v.at[0]], buf, sem).wait()

    def body(jj, carry):
        j0 = 2 * jj
        pltpu.async_copy(m_hbm.at[src_v.at[j0 + 1]], rows1, sem1)
        wait_gather(rows0, sem0)
        pltpu.sync_copy(rows0, acc.at[dst_v.at[j0]], add=True)
        jn = jnp.minimum(j0 + 2, NCHUNK - 1)
        pltpu.async_copy(m_hbm.at[src_v.at[jn]], rows0, sem0)
        wait_gather(rows1, sem1)
        pltpu.sync_copy(rows1, acc.at[dst_v.at[j0 + 1]], add=True)
        return carry

    lax.fori_loop(0, NCHUNK // 2, body, 0)
    wait_gather(rows0, sem0)
    plsc.subcore_barrier()
    pltpu.sync_copy(acc.at[pl.ds(s * ROWS_PER_TILE, ROWS_PER_TILE)],
                    part_hbm.at[c, pl.ds(s * ROWS_PER_TILE, ROWS_PER_TILE)])


@functools.cache
def _make_sc_scatter():
    return functools.partial(
        pl.kernel,
        out_type=jax.ShapeDtypeStruct((NC, ACC_ROWS, VW), jnp.float32),
        mesh=plsc.VectorSubcoreMesh(core_axis_name="c", subcore_axis_name="s",
                                    num_cores=NC, num_subcores=NS),
        scratch_types=[
            pltpu.VMEM((NCHUNK, CHUNK), jnp.int32),
            pltpu.VMEM((NCHUNK, CHUNK), jnp.int32),
            pltpu.VMEM((CHUNK, VW), jnp.float32),
            pltpu.VMEM((CHUNK, VW), jnp.float32),
            pltpu.VMEM_SHARED((ACC_ROWS, VW), jnp.float32),
            pltpu.SemaphoreType.DMA,
            pltpu.SemaphoreType.DMA,
        ],
    )(_sc_scatter_body)


def _dot(a, b):
    return jnp.dot(a, b, preferred_element_type=jnp.float32)


def _msg_and_proj(h, wg, bg, wm, wp, bp, m_ref, inp_ref):
    m = jax.nn.sigmoid(_dot(h, wg[...]) + bg[...]) * _dot(h, wm[...])
    m_ref[...] = jnp.concatenate(
        [m, jnp.zeros((m.shape[0], VW - VHS), jnp.float32)], axis=1)
    inp_ref[...] = _dot(h, wp[...]) + bp[...]


def _tc_round0_body(x_ref, wir, wiz, win, bir, biz, bin_, bhr, bhz, bhn,
                    wg, bg, wm, wp, bp, m_ref, inp_ref):
    x = x_ref[...]
    ir = _dot(x, wir[...]) + bir[...]
    iz = _dot(x, wiz[...]) + biz[...]
    inn = _dot(x, win[...]) + bin_[...]
    r = jax.nn.sigmoid(ir + bhr[...])
    z = jax.nn.sigmoid(iz + bhz[...])
    n = jnp.tanh(inn + r * bhn[...])
    h = (1.0 - z) * n
    _msg_and_proj(h, wg, bg, wm, wp, bp, m_ref, inp_ref)


def _gru_from_parts(inp_ref, p0_ref, p1_ref, wir, wiz, win, bir, biz, bin_,
                    whr, whz, whn, bhr, bhz, bhn):
    ps = p0_ref[0] + p1_ref[0]
    inp = inp_ref[...]
    ir = _dot(inp, wir[...]) + bir[...]
    iz = _dot(inp, wiz[...]) + biz[...]
    inn = _dot(inp, win[...]) + bin_[...]
    hr = _dot(ps, whr[...]) + bhr[...]
    hz = _dot(ps, whz[...]) + bhz[...]
    hn = _dot(ps, whn[...]) + bhn[...]
    r = jax.nn.sigmoid(ir + hr)
    z = jax.nn.sigmoid(iz + hz)
    n = jnp.tanh(inn + r * hn)
    return (1.0 - z) * n + z * ps[:, :VHS]


def _tc_round_body(inp_ref, p0_ref, p1_ref, wir, wiz, win, bir, biz, bin_,
                   whr, whz, whn, bhr, bhz, bhn, wg, bg, wm, wp, bp,
                   m_ref, inp2_ref):
    h = _gru_from_parts(inp_ref, p0_ref, p1_ref, wir, wiz, win, bir, biz,
                        bin_, whr, whz, whn, bhr, bhz, bhn)
    _msg_and_proj(h, wg, bg, wm, wp, bp, m_ref, inp2_ref)


def _tc_final_body(inp_ref, p0_ref, p1_ref, wir, wiz, win, bir, biz, bin_,
                   whr, whz, whn, bhr, bhz, bhn, wc1, bc1, wc2, bc2, out_ref):
    h = _gru_from_parts(inp_ref, p0_ref, p1_ref, wir, wiz, win, bir, biz,
                        bin_, whr, whz, whn, bhr, bhz, bhn)
    hid = jax.nn.relu(_dot(h, wc1[...]) + bc1[...])
    out_ref[...] = jax.nn.sigmoid(_dot(hid, wc2[...]) + bc2[...])


def _full(shape):
    return pl.BlockSpec(shape, lambda i: (0,) * len(shape))


def _rows(width):
    return pl.BlockSpec((BR, width), lambda i: (i, 0))


def _part_spec(core):
    return pl.BlockSpec((1, BR, VW), lambda i, c=core: (c, i, 0))


def kernel(x, edge_index, W_ih, b_ih, W_hh, b_hh, Wg, bg, Wm, Wp, bp,
           Wc1, bc1, Wc2, bc2):
    f32 = jnp.float32

    wihT = W_ih.T
    wir, wiz, win = (wihT[:, :VHS], wihT[:, VHS:2 * VHS], wihT[:, 2 * VHS:])
    whhT = W_hh.T
    pad_w = ((0, VW - VHS), (0, 0))
    whr = jnp.pad(whhT[:, :VHS], pad_w)
    whz = jnp.pad(whhT[:, VHS:2 * VHS], pad_w)
    whn = jnp.pad(whhT[:, 2 * VHS:], pad_w)
    bir, biz, bin_ = (b_ih[:VHS].reshape(1, VHS),
                      b_ih[VHS:2 * VHS].reshape(1, VHS),
                      b_ih[2 * VHS:].reshape(1, VHS))
    bhr, bhz, bhn = (b_hh[:VHS].reshape(1, VHS),
                     b_hh[VHS:2 * VHS].reshape(1, VHS),
                     b_hh[2 * VHS:].reshape(1, VHS))
    wg, wm, wp = Wg.T, Wm.T, Wp.T
    bg2, bp2 = bg.reshape(1, VHS), bp.reshape(1, NVT)
    wc1, wc2 = Wc1.T, Wc2.T
    bc12, bc22 = bc1.reshape(1, CHS), bc2.reshape(1, 1)

    pad_e = E_PAD - N_EDGES
    src3 = jnp.concatenate(
        [edge_index[0], jnp.zeros((pad_e,), jnp.int32)]).reshape(
            NW, NCHUNK, CHUNK)
    dst3 = jnp.concatenate(
        [edge_index[1], jnp.full((pad_e,), N_NODES, jnp.int32)]).reshape(
            NW, NCHUNK, CHUNK)
    zeros_tile = jnp.zeros((ROWS_PER_TILE, VW), f32)

    gru_w = (wir, wiz, win, bir, biz, bin_)
    gru_h = (whr, whz, whn, bhr, bhz, bhn)
    msg_w = (wg, bg2, wm, wp, bp2)

    gru_w_specs = [_full((NVT, VHS))] * 3 + [_full((1, VHS))] * 3
    gru_h_specs = [_full((VW, VHS))] * 3 + [_full((1, VHS))] * 3
    msg_specs = [_full((VHS, VHS)), _full((1, VHS)), _full((VHS, VHS)),
                 _full((VHS, NVT)), _full((1, NVT))]

    m_shape = jax.ShapeDtypeStruct((N_NODES, VW), f32)
    inp_shape = jax.ShapeDtypeStruct((N_NODES, NVT), f32)

    m1, inp1 = pl.pallas_call(
        _tc_round0_body,
        grid=(GRID,),
        in_specs=[_rows(NVT)] + gru_w_specs + gru_h_specs[3:] + msg_specs,
        out_specs=[_rows(VW), _rows(NVT)],
        out_shape=[m_shape, inp_shape],
    )(x, *gru_w, *gru_h[3:], *msg_w)

    sc_scatter = _make_sc_scatter()
    parts1 = sc_scatter(m1, src3, dst3, zeros_tile)
    m2, inp2 = pl.pallas_call(
        _tc_round_body,
        grid=(GRID,),
        in_specs=([_rows(NVT), _part_spec(0), _part_spec(1)]
                  + gru_w_specs + gru_h_specs + msg_specs),
        out_specs=[_rows(VW), _rows(NVT)],
        out_shape=[m_shape, inp_shape],
    )(inp1, parts1, parts1, *gru_w, *gru_h, *msg_w)

    parts2 = sc_scatter(m2, src3, dst3, zeros_tile)
    out = pl.pallas_call(
        _tc_final_body,
        grid=(GRID,),
        in_specs=([_rows(NVT), _part_spec(0), _part_spec(1)]
                  + gru_w_specs + gru_h_specs
                  + [_full((VHS, CHS)), _full((1, CHS)),
                     _full((CHS, 1)), _full((1, 1))]),
        out_specs=[_rows(1)],
        out_shape=[jax.ShapeDtypeStruct((N_NODES, 1), f32)],
    )(inp2, parts2, parts2, *gru_w, *gru_h, wc1, bc12, wc2, bc22)[0]

    return out


# speedup vs baseline: 4.5036x; 1.0071x over previous
import functools

import jax
import jax.numpy as jnp
from jax import lax
from jax.experimental import pallas as pl
from jax.experimental.pallas import tpu as pltpu
from jax.experimental.pallas import tpu_sc as plsc

N_NODES = 10000
N_EDGES = 160000
VHS = 100
VW = 128
NVT = 3
CHS = 30

NC = 2
NS = 16
NW = NC * NS
CHUNK = 128
NCHUNK = (N_EDGES + NW * CHUNK - 1) // (NW * CHUNK)
E_PAD = NW * NCHUNK * CHUNK
ROWS_PER_TILE = 632
ACC_ROWS = NS * ROWS_PER_TILE

BR = 1000
GRID = N_NODES // BR


NBUF = 2


def _sc_scatter_body(m_hbm, src_hbm, dst_hbm, zeros_hbm, part_hbm,
                     src_v, dst_v, rows, acc, gsems, ssems):
    c = lax.axis_index("c")
    s = lax.axis_index("s")
    wid = s * NC + c

    pltpu.sync_copy(src_hbm.at[wid], src_v)
    pltpu.sync_copy(dst_hbm.at[wid], dst_v)
    pltpu.sync_copy(zeros_hbm, acc.at[pl.ds(s * ROWS_PER_TILE, ROWS_PER_TILE)])
    plsc.subcore_barrier()

    for k in range(NBUF):
        pltpu.async_copy(m_hbm.at[src_v.at[k]], rows[k], gsems[k])

    def quad(j0, refill):
        for k in range(NBUF):
            pltpu.make_async_copy(m_hbm.at[src_v.at[0]], rows[k],
                                  gsems[k]).wait()
            pltpu.async_copy(rows[k], acc.at[dst_v.at[j0 + k]], ssems[k],
                             add=True)
        for k in range(NBUF):
            pltpu.make_async_copy(rows[k], acc.at[dst_v.at[0]],
                                  ssems[k]).wait()
            if refill:
                pltpu.async_copy(m_hbm.at[src_v.at[j0 + NBUF + k]], rows[k],
                                 gsems[k])

    def body(jj, carry):
        quad(NBUF * jj, True)
        return carry

    lax.fori_loop(0, NCHUNK // NBUF - 1, body, 0)
    quad(NCHUNK - NBUF, False)
    plsc.subcore_barrier()
    pltpu.sync_copy(acc.at[pl.ds(s * ROWS_PER_TILE, ROWS_PER_TILE)],
                    part_hbm.at[c, pl.ds(s * ROWS_PER_TILE, ROWS_PER_TILE)])


@functools.cache
def _make_sc_scatter():
    return functools.partial(
        pl.kernel,
        out_type=jax.ShapeDtypeStruct((NC, ACC_ROWS, VW), jnp.float32),
        mesh=plsc.VectorSubcoreMesh(core_axis_name="c", subcore_axis_name="s",
                                    num_cores=NC, num_subcores=NS),
        scratch_types=[
            pltpu.VMEM((NCHUNK, CHUNK), jnp.int32),
            pltpu.VMEM((NCHUNK, CHUNK), jnp.int32),
            [pltpu.VMEM((CHUNK, VW), jnp.float32)] * NBUF,
            pltpu.VMEM_SHARED((ACC_ROWS, VW), jnp.float32),
            [pltpu.SemaphoreType.DMA] * NBUF,
            [pltpu.SemaphoreType.DMA] * NBUF,
        ],
    )(_sc_scatter_body)


def _dot(a, b):
    return jnp.dot(a, b, preferred_element_type=jnp.float32)


def _msg_and_proj(h, wg, bg, wm, wp, bp, m_ref, inp_ref):
    m = jax.nn.sigmoid(_dot(h, wg[...]) + bg[...]) * _dot(h, wm[...])
    m_ref[...] = jnp.concatenate(
        [m, jnp.zeros((m.shape[0], VW - VHS), jnp.float32)], axis=1)
    inp_ref[...] = _dot(h, wp[...]) + bp[...]


def _tc_round0_body(x_ref, wir, wiz, win, bir, biz, bin_, bhr, bhz, bhn,
                    wg, bg, wm, wp, bp, m_ref, inp_ref):
    x = x_ref[...]
    ir = _dot(x, wir[...]) + bir[...]
    iz = _dot(x, wiz[...]) + biz[...]
    inn = _dot(x, win[...]) + bin_[...]
    r = jax.nn.sigmoid(ir + bhr[...])
    z = jax.nn.sigmoid(iz + bhz[...])
    n = jnp.tanh(inn + r * bhn[...])
    h = (1.0 - z) * n
    _msg_and_proj(h, wg, bg, wm, wp, bp, m_ref, inp_ref)


def _gru_from_parts(inp_ref, p0_ref, p1_ref, wir, wiz, win, bir, biz, bin_,
                    whr, whz, whn, bhr, bhz, bhn):
    ps = p0_ref[0] + p1_ref[0]
    inp = inp_ref[...]
    ir = _dot(inp, wir[...]) + bir[...]
    iz = _dot(inp, wiz[...]) + biz[...]
    inn = _dot(inp, win[...]) + bin_[...]
    hr = _dot(ps, whr[...]) + bhr[...]
    hz = _dot(ps, whz[...]) + bhz[...]
    hn = _dot(ps, whn[...]) + bhn[...]
    r = jax.nn.sigmoid(ir + hr)
    z = jax.nn.sigmoid(iz + hz)
    n = jnp.tanh(inn + r * hn)
    return (1.0 - z) * n + z * ps[:, :VHS]


def _tc_round_body(inp_ref, p0_ref, p1_ref, wir, wiz, win, bir, biz, bin_,
                   whr, whz, whn, bhr, bhz, bhn, wg, bg, wm, wp, bp,
                   m_ref, inp2_ref):
    h = _gru_from_parts(inp_ref, p0_ref, p1_ref, wir, wiz, win, bir, biz,
                        bin_, whr, whz, whn, bhr, bhz, bhn)
    _msg_and_proj(h, wg, bg, wm, wp, bp, m_ref, inp2_ref)


def _tc_final_body(inp_ref, p0_ref, p1_ref, wir, wiz, win, bir, biz, bin_,
                   whr, whz, whn, bhr, bhz, bhn, wc1, bc1, wc2, bc2, out_ref):
    h = _gru_from_parts(inp_ref, p0_ref, p1_ref, wir, wiz, win, bir, biz,
                        bin_, whr, whz, whn, bhr, bhz, bhn)
    hid = jax.nn.relu(_dot(h, wc1[...]) + bc1[...])
    out_ref[...] = jax.nn.sigmoid(_dot(hid, wc2[...]) + bc2[...])


def _full(shape):
    return pl.BlockSpec(shape, lambda i: (0,) * len(shape))


def _rows(width):
    return pl.BlockSpec((BR, width), lambda i: (i, 0))


def _part_spec(core):
    return pl.BlockSpec((1, BR, VW), lambda i, c=core: (c, i, 0))


def kernel(x, edge_index, W_ih, b_ih, W_hh, b_hh, Wg, bg, Wm, Wp, bp,
           Wc1, bc1, Wc2, bc2):
    f32 = jnp.float32

    wihT = W_ih.T
    wir, wiz, win = (wihT[:, :VHS], wihT[:, VHS:2 * VHS], wihT[:, 2 * VHS:])
    whhT = W_hh.T
    pad_w = ((0, VW - VHS), (0, 0))
    whr = jnp.pad(whhT[:, :VHS], pad_w)
    whz = jnp.pad(whhT[:, VHS:2 * VHS], pad_w)
    whn = jnp.pad(whhT[:, 2 * VHS:], pad_w)
    bir, biz, bin_ = (b_ih[:VHS].reshape(1, VHS),
                      b_ih[VHS:2 * VHS].reshape(1, VHS),
                      b_ih[2 * VHS:].reshape(1, VHS))
    bhr, bhz, bhn = (b_hh[:VHS].reshape(1, VHS),
                     b_hh[VHS:2 * VHS].reshape(1, VHS),
                     b_hh[2 * VHS:].reshape(1, VHS))
    wg, wm, wp = Wg.T, Wm.T, Wp.T
    bg2, bp2 = bg.reshape(1, VHS), bp.reshape(1, NVT)
    wc1, wc2 = Wc1.T, Wc2.T
    bc12, bc22 = bc1.reshape(1, CHS), bc2.reshape(1, 1)

    pad_e = E_PAD - N_EDGES
    src3 = jnp.concatenate(
        [edge_index[0], jnp.zeros((pad_e,), jnp.int32)]).reshape(
            NW, NCHUNK, CHUNK)
    dst3 = jnp.concatenate(
        [edge_index[1], jnp.full((pad_e,), N_NODES, jnp.int32)]).reshape(
            NW, NCHUNK, CHUNK)
    zeros_tile = jnp.zeros((ROWS_PER_TILE, VW), f32)

    gru_w = (wir, wiz, win, bir, biz, bin_)
    gru_h = (whr, whz, whn, bhr, bhz, bhn)
    msg_w = (wg, bg2, wm, wp, bp2)

    gru_w_specs = [_full((NVT, VHS))] * 3 + [_full((1, VHS))] * 3
    gru_h_specs = [_full((VW, VHS))] * 3 + [_full((1, VHS))] * 3
    msg_specs = [_full((VHS, VHS)), _full((1, VHS)), _full((VHS, VHS)),
                 _full((VHS, NVT)), _full((1, NVT))]

    m_shape = jax.ShapeDtypeStruct((N_NODES, VW), f32)
    inp_shape = jax.ShapeDtypeStruct((N_NODES, NVT), f32)

    m1, inp1 = pl.pallas_call(
        _tc_round0_body,
        grid=(GRID,),
        in_specs=[_rows(NVT)] + gru_w_specs + gru_h_specs[3:] + msg_specs,
        out_specs=[_rows(VW), _rows(NVT)],
        out_shape=[m_shape, inp_shape],
    )(x, *gru_w, *gru_h[3:], *msg_w)

    sc_scatter = _make_sc_scatter()
    parts1 = sc_scatter(m1, src3, dst3, zeros_tile)
    m2, inp2 = pl.pallas_call(
        _tc_round_body,
        grid=(GRID,),
        in_specs=([_rows(NVT), _part_spec(0), _part_spec(1)]
                  + gru_w_specs + gru_h_specs + msg_specs),
        out_specs=[_rows(VW), _rows(NVT)],
        out_shape=[m_shape, inp_shape],
    )(inp1, parts1, parts1, *gru_w, *gru_h, *msg_w)

    parts2 = sc_scatter(m2, src3, dst3, zeros_tile)
    out = pl.pallas_call(
        _tc_final_body,
        grid=(GRID,),
        in_specs=([_rows(NVT), _part_spec(0), _part_spec(1)]
                  + gru_w_specs + gru_h_specs
                  + [_full((VHS, CHS)), _full((1, CHS)),
                     _full((CHS, 1)), _full((1, 1))]),
        out_specs=[_rows(1)],
        out_shape=[jax.ShapeDtypeStruct((N_NODES, 1), f32)],
    )(inp2, parts2, parts2, *gru_w, *gru_h, wc1, bc12, wc2, bc22)[0]

    return out


# speedup vs baseline: 4.5979x; 1.0210x over previous
import functools

import jax
import jax.numpy as jnp
from jax import lax
from jax.experimental import pallas as pl
from jax.experimental.pallas import tpu as pltpu
from jax.experimental.pallas import tpu_sc as plsc

N_NODES = 10000
N_EDGES = 160000
VHS = 100
VW = 128
NVT = 3
CHS = 30

NC = 2
NS = 16
NW = NC * NS
CHUNK = 128
NCHUNK = (N_EDGES + NW * CHUNK - 1) // (NW * CHUNK)
E_PAD = NW * NCHUNK * CHUNK
ROWS_PER_TILE = 640
ACC_ROWS = NS * ROWS_PER_TILE

BR = 2000
GRID = N_NODES // BR


NBUF = 2


def _sc_scatter_body(m_hbm, src_hbm, dst_hbm, zeros_hbm, part_hbm,
                     src_v, dst_v, rows, acc, gsems, ssems):
    c = lax.axis_index("c")
    s = lax.axis_index("s")
    wid = s * NC + c

    pltpu.sync_copy(src_hbm.at[wid], src_v)
    pltpu.sync_copy(dst_hbm.at[wid], dst_v)
    pltpu.sync_copy(zeros_hbm, acc.at[pl.ds(s * ROWS_PER_TILE, ROWS_PER_TILE)])
    plsc.subcore_barrier()

    for k in range(NBUF):
        pltpu.async_copy(m_hbm.at[src_v.at[k]], rows[k], gsems[k])

    def quad(j0, refill):
        for k in range(NBUF):
            pltpu.make_async_copy(m_hbm.at[src_v.at[0]], rows[k],
                                  gsems[k]).wait()
            pltpu.async_copy(rows[k], acc.at[dst_v.at[j0 + k]], ssems[k],
                             add=True)
        for k in range(NBUF):
            pltpu.make_async_copy(rows[k], acc.at[dst_v.at[0]],
                                  ssems[k]).wait()
            if refill:
                pltpu.async_copy(m_hbm.at[src_v.at[j0 + NBUF + k]], rows[k],
                                 gsems[k])

    def body(jj, carry):
        quad(NBUF * jj, True)
        return carry

    lax.fori_loop(0, NCHUNK // NBUF - 1, body, 0)
    quad(NCHUNK - NBUF, False)
    plsc.subcore_barrier()
    pltpu.sync_copy(acc.at[pl.ds(s * ROWS_PER_TILE, ROWS_PER_TILE)],
                    part_hbm.at[c, pl.ds(s * ROWS_PER_TILE, ROWS_PER_TILE)])


@functools.cache
def _make_sc_scatter():
    return functools.partial(
        pl.kernel,
        out_type=jax.ShapeDtypeStruct((NC, ACC_ROWS, VW), jnp.float32),
        mesh=plsc.VectorSubcoreMesh(core_axis_name="c", subcore_axis_name="s",
                                    num_cores=NC, num_subcores=NS),
        scratch_types=[
            pltpu.VMEM((NCHUNK, CHUNK), jnp.int32),
            pltpu.VMEM((NCHUNK, CHUNK), jnp.int32),
            [pltpu.VMEM((CHUNK, VW), jnp.float32)] * NBUF,
            pltpu.VMEM_SHARED((ACC_ROWS, VW), jnp.float32),
            [pltpu.SemaphoreType.DMA] * NBUF,
            [pltpu.SemaphoreType.DMA] * NBUF,
        ],
    )(_sc_scatter_body)


def _dot(a, b):
    return jnp.dot(a, b, preferred_element_type=jnp.float32)


def _msg_and_proj(h, wg, bg, wm, wp, bp, m_ref, inp_ref):
    m = jax.nn.sigmoid(_dot(h, wg[...]) + bg[...]) * _dot(h, wm[...])
    m_ref[...] = jnp.concatenate(
        [m, jnp.zeros((m.shape[0], VW - VHS), jnp.float32)],
        axis=1)
    inp_ref[...] = _dot(h, wp[...]) + bp[...]


def _tc_round0_body(x_ref, wir, wiz, win, bir, biz, bin_, bhr, bhz, bhn,
                    wg, bg, wm, wp, bp, m_ref, inp_ref):
    x = x_ref[...]
    ir = _dot(x, wir[...]) + bir[...]
    iz = _dot(x, wiz[...]) + biz[...]
    inn = _dot(x, win[...]) + bin_[...]
    r = jax.nn.sigmoid(ir + bhr[...])
    z = jax.nn.sigmoid(iz + bhz[...])
    n = jnp.tanh(inn + r * bhn[...])
    h = (1.0 - z) * n
    _msg_and_proj(h, wg, bg, wm, wp, bp, m_ref, inp_ref)


def _gru_from_parts(inp_ref, p0_ref, p1_ref, wir, wiz, win, bir, biz, bin_,
                    whr, whz, whn, bhr, bhz, bhn):
    ps = (p0_ref[0].astype(jnp.float32)
          + p1_ref[0].astype(jnp.float32))
    inp = inp_ref[...]
    ir = _dot(inp, wir[...]) + bir[...]
    iz = _dot(inp, wiz[...]) + biz[...]
    inn = _dot(inp, win[...]) + bin_[...]
    hr = _dot(ps, whr[...]) + bhr[...]
    hz = _dot(ps, whz[...]) + bhz[...]
    hn = _dot(ps, whn[...]) + bhn[...]
    r = jax.nn.sigmoid(ir + hr)
    z = jax.nn.sigmoid(iz + hz)
    n = jnp.tanh(inn + r * hn)
    return (1.0 - z) * n + z * ps[:, :VHS]


def _tc_round_body(inp_ref, p0_ref, p1_ref, wir, wiz, win, bir, biz, bin_,
                   whr, whz, whn, bhr, bhz, bhn, wg, bg, wm, wp, bp,
                   m_ref, inp2_ref):
    h = _gru_from_parts(inp_ref, p0_ref, p1_ref, wir, wiz, win, bir, biz,
                        bin_, whr, whz, whn, bhr, bhz, bhn)
    _msg_and_proj(h, wg, bg, wm, wp, bp, m_ref, inp2_ref)


def _tc_final_body(inp_ref, p0_ref, p1_ref, wir, wiz, win, bir, biz, bin_,
                   whr, whz, whn, bhr, bhz, bhn, wc1, bc1, wc2, bc2, out_ref):
    h = _gru_from_parts(inp_ref, p0_ref, p1_ref, wir, wiz, win, bir, biz,
                        bin_, whr, whz, whn, bhr, bhz, bhn)
    hid = jax.nn.relu(_dot(h, wc1[...]) + bc1[...])
    out_ref[...] = jax.nn.sigmoid(_dot(hid, wc2[...]) + bc2[...])


def _full(shape):
    return pl.BlockSpec(shape, lambda i: (0,) * len(shape))


def _rows(width):
    return pl.BlockSpec((BR, width), lambda i: (i, 0))


def _part_spec(core):
    return pl.BlockSpec((1, BR, VW), lambda i, c=core: (c, i, 0))


def kernel(x, edge_index, W_ih, b_ih, W_hh, b_hh, Wg, bg, Wm, Wp, bp,
           Wc1, bc1, Wc2, bc2):
    f32 = jnp.float32

    wihT = W_ih.T
    wir, wiz, win = (wihT[:, :VHS], wihT[:, VHS:2 * VHS], wihT[:, 2 * VHS:])
    whhT = W_hh.T
    pad_w = ((0, VW - VHS), (0, 0))
    whr = jnp.pad(whhT[:, :VHS], pad_w)
    whz = jnp.pad(whhT[:, VHS:2 * VHS], pad_w)
    whn = jnp.pad(whhT[:, 2 * VHS:], pad_w)
    bir, biz, bin_ = (b_ih[:VHS].reshape(1, VHS),
                      b_ih[VHS:2 * VHS].reshape(1, VHS),
                      b_ih[2 * VHS:].reshape(1, VHS))
    bhr, bhz, bhn = (b_hh[:VHS].reshape(1, VHS),
                     b_hh[VHS:2 * VHS].reshape(1, VHS),
                     b_hh[2 * VHS:].reshape(1, VHS))
    wg, wm, wp = Wg.T, Wm.T, Wp.T
    bg2, bp2 = bg.reshape(1, VHS), bp.reshape(1, NVT)
    wc1, wc2 = Wc1.T, Wc2.T
    bc12, bc22 = bc1.reshape(1, CHS), bc2.reshape(1, 1)

    pad_e = E_PAD - N_EDGES
    src3 = jnp.concatenate(
        [edge_index[0], jnp.zeros((pad_e,), jnp.int32)]).reshape(
            NW, NCHUNK, CHUNK)
    dst3 = jnp.concatenate(
        [edge_index[1], jnp.full((pad_e,), N_NODES, jnp.int32)]).reshape(
            NW, NCHUNK, CHUNK)
    zeros_tile = jnp.zeros((ROWS_PER_TILE, VW), jnp.float32)

    gru_w = (wir, wiz, win, bir, biz, bin_)
    gru_h = (whr, whz, whn, bhr, bhz, bhn)
    msg_w = (wg, bg2, wm, wp, bp2)

    gru_w_specs = [_full((NVT, VHS))] * 3 + [_full((1, VHS))] * 3
    gru_h_specs = [_full((VW, VHS))] * 3 + [_full((1, VHS))] * 3
    msg_specs = [_full((VHS, VHS)), _full((1, VHS)), _full((VHS, VHS)),
                 _full((VHS, NVT)), _full((1, NVT))]

    m_shape = jax.ShapeDtypeStruct((N_NODES, VW), jnp.float32)

    inp_shape = jax.ShapeDtypeStruct((N_NODES, NVT), f32)

    m1, inp1 = pl.pallas_call(
        _tc_round0_body,
        grid=(GRID,),
        in_specs=[_rows(NVT)] + gru_w_specs + gru_h_specs[3:] + msg_specs,
        out_specs=[_rows(VW), _rows(NVT)],
        out_shape=[m_shape, inp_shape],
    )(x, *gru_w, *gru_h[3:], *msg_w)

    sc_scatter = _make_sc_scatter()
    parts1 = sc_scatter(m1, src3, dst3, zeros_tile)
    m2, inp2 = pl.pallas_call(
        _tc_round_body,
        grid=(GRID,),
        in_specs=([_rows(NVT), _part_spec(0), _part_spec(1)]
                  + gru_w_specs + gru_h_specs + msg_specs),
        out_specs=[_rows(VW), _rows(NVT)],
        out_shape=[m_shape, inp_shape],
    )(inp1, parts1, parts1, *gru_w, *gru_h, *msg_w)

    parts2 = sc_scatter(m2, src3, dst3, zeros_tile)
    out = pl.pallas_call(
        _tc_final_body,
        grid=(GRID,),
        in_specs=([_rows(NVT), _part_spec(0), _part_spec(1)]
                  + gru_w_specs + gru_h_specs
                  + [_full((VHS, CHS)), _full((1, CHS)),
                     _full((CHS, 1)), _full((1, 1))]),
        out_specs=[_rows(1)],
        out_shape=[jax.ShapeDtypeStruct((N_NODES, 1), f32)],
    )(inp2, parts2, parts2, *gru_w, *gru_h, wc1, bc12, wc2, bc22)[0]

    return out
